# speedup vs baseline: 3.0910x; 3.0910x over previous
import jax
import jax.numpy as jnp
from jax.experimental import pallas as pl
from jax.experimental.pallas import tpu as pltpu
from jax.experimental.pallas import tpu_sc as plsc

_WINDOW = 128


def kernel(X, table):
    B, H = X.shape
    V, D = table.shape
    n = B * H
    idx = X.reshape(1, n).astype(jnp.int32)

    mesh = plsc.VectorSubcoreMesh(
        core_axis_name="core", subcore_axis_name="subcore"
    )

    @pl.kernel(
        out_type=jax.ShapeDtypeStruct((n, D), table.dtype),
        mesh=mesh,
    )
    def gather_kernel(tab_hbm, idx_hbm, out_hbm):
        def body(i_vmem, o_vmem):
            pltpu.sync_copy(tab_hbm.at[i_vmem.at[0]], o_vmem)

        pltpu.emit_pipeline(
            body,
            grid=(n // _WINDOW,),
            in_specs=[
                pl.BlockSpec((1, _WINDOW), index_map=lambda i: (0, i))
            ],
            out_specs=[
                pl.BlockSpec((_WINDOW, D), index_map=lambda i: (i, 0))
            ],
            core_axis_name=("core", "subcore"),
            dimension_semantics=(pltpu.PARALLEL,),
        )(idx_hbm, out_hbm)

    out = gather_kernel(table, idx)
    return out.reshape(B, H, D)


# speedup vs baseline: 3.3441x; 1.0819x over previous
import jax
import jax.numpy as jnp
from jax import lax
from jax.experimental import pallas as pl
from jax.experimental.pallas import tpu as pltpu
from jax.experimental.pallas import tpu_sc as plsc

_NC = 2
_NS = 16
_NW = _NC * _NS
_WIN = 128
_GROUP = 256
_SUB = _GROUP // _WIN


def kernel(X, table):
    B, H = X.shape
    V, D = table.shape
    n = B * H
    rows_per_w = n // _NW
    wins_per_w = rows_per_w // _WIN
    ngroups = rows_per_w // _GROUP
    assert n % (_NW * _GROUP) == 0 and ngroups % 2 == 1

    idx = X.reshape(n).astype(jnp.int32)

    mesh = plsc.VectorSubcoreMesh(core_axis_name="c", subcore_axis_name="s")

    @pl.kernel(
        out_type=jax.ShapeDtypeStruct((n, D), table.dtype),
        mesh=mesh,
        scratch_types=[
            pltpu.VMEM((rows_per_w,), jnp.int32),
            pltpu.VMEM((_GROUP, D), table.dtype),
            pltpu.VMEM((_GROUP, D), table.dtype),
            pltpu.SemaphoreType.DMA,
            pltpu.SemaphoreType.DMA,
            pltpu.SemaphoreType.DMA,
            pltpu.SemaphoreType.DMA,
        ],
    )
    def gather_kernel(tab_hbm, idx_hbm, out_hbm,
                      idx_v, buf_a, buf_b, g_a, g_b, o_a, o_b):
        wid = lax.axis_index("c") * _NS + lax.axis_index("s")
        rowbase = wid * rows_per_w

        pltpu.sync_copy(idx_hbm.at[pl.ds(rowbase, rows_per_w)], idx_v)

        def fire_gather(g, buf, sem):
            pltpu.async_copy(
                tab_hbm.at[idx_v.at[pl.ds(g * _GROUP, _WIN)]],
                buf.at[pl.ds(0, _WIN)], sem)
            pltpu.async_copy(
                tab_hbm.at[idx_v.at[pl.ds(g * _GROUP + _WIN, _WIN)]],
                buf.at[pl.ds(_WIN, _WIN)], sem)

        def wait_gather(buf, sem):
            pltpu.make_async_copy(
                tab_hbm.at[pl.ds(0, _GROUP)], buf, sem).wait()

        def fire_out(g, buf, sem):
            pltpu.async_copy(
                buf, out_hbm.at[pl.ds(rowbase + g * _GROUP, _GROUP)], sem)

        def wait_out(g, buf, sem):
            pltpu.make_async_copy(
                buf, out_hbm.at[pl.ds(rowbase + g * _GROUP, _GROUP)],
                sem).wait()

        fire_gather(0, buf_a, g_a)
        fire_gather(1, buf_b, g_b)

        @pl.loop(0, ngroups - 1, step=2)
        def _(g0):
            wait_gather(buf_a, g_a)
            fire_out(g0, buf_a, o_a)
            wait_out(g0, buf_a, o_a)
            fire_gather(g0 + 2, buf_a, g_a)
            wait_gather(buf_b, g_b)
            fire_out(g0 + 1, buf_b, o_b)

            @pl.when(g0 + 3 < ngroups)
            def _():
                wait_out(g0 + 1, buf_b, o_b)
                fire_gather(g0 + 3, buf_b, g_b)

        g_last = ngroups - 1
        wait_gather(buf_a, g_a)
        fire_out(g_last, buf_a, o_a)
        wait_out(g_last, buf_a, o_a)
        wait_out(g_last - 1, buf_b, o_b)

    out = gather_kernel(table, idx)
    return out.reshape(B, H, D)


# speedup vs baseline: 5.9371x; 1.7754x over previous
import jax
import jax.numpy as jnp
from jax import lax
from jax.experimental import pallas as pl
from jax.experimental.pallas import tpu as pltpu
from jax.experimental.pallas import tpu_sc as plsc

_NC = 2
_NS = 16
_NW = _NC * _NS
_RPG = 8


def kernel(X, table):
    B, H = X.shape
    V, D = table.shape
    rows_per_w = B // _NW
    ngroups = rows_per_w // _RPG
    assert B % (_NW * _RPG) == 0 and ngroups % 2 == 0

    Xi = X.astype(jnp.int32)

    mesh = plsc.VectorSubcoreMesh(core_axis_name="c", subcore_axis_name="s")

    @pl.kernel(
        out_type=jax.ShapeDtypeStruct((B, H, D), table.dtype),
        mesh=mesh,
        scratch_types=[
            pltpu.VMEM((rows_per_w, H), jnp.int32),
            pltpu.VMEM((_RPG, H, D), table.dtype),
            pltpu.VMEM((_RPG, H, D), table.dtype),
            pltpu.SemaphoreType.DMA,
            pltpu.SemaphoreType.DMA,
            pltpu.SemaphoreType.DMA,
            pltpu.SemaphoreType.DMA,
        ],
    )
    def gather_kernel(tab_hbm, idx_hbm, out_hbm,
                      idx_v, buf_a, buf_b, g_a, g_b, o_a, o_b):
        wid = lax.axis_index("c") * _NS + lax.axis_index("s")
        rowbase = wid * rows_per_w

        pltpu.sync_copy(idx_hbm.at[pl.ds(rowbase, rows_per_w)], idx_v)

        def fire_gather(g, buf, sem):
            for i in range(_RPG):
                pltpu.async_copy(
                    tab_hbm.at[idx_v.at[g * _RPG + i]], buf.at[i], sem)

        def wait_gather(buf, sem):
            pltpu.make_async_copy(out_hbm.at[pl.ds(0, _RPG)], buf, sem).wait()

        def fire_out(g, buf, sem):
            pltpu.async_copy(
                buf, out_hbm.at[pl.ds(rowbase + g * _RPG, _RPG)], sem)

        def wait_out(g, buf, sem):
            pltpu.make_async_copy(
                buf, out_hbm.at[pl.ds(rowbase + g * _RPG, _RPG)], sem).wait()

        fire_gather(0, buf_a, g_a)
        fire_gather(1, buf_b, g_b)

        @pl.loop(0, ngroups, step=2)
        def _(g0):
            wait_gather(buf_a, g_a)
            fire_out(g0, buf_a, o_a)
            wait_out(g0, buf_a, o_a)

            @pl.when(g0 + 2 < ngroups)
            def _():
                fire_gather(g0 + 2, buf_a, g_a)

            wait_gather(buf_b, g_b)
            fire_out(g0 + 1, buf_b, o_b)

            @pl.when(g0 + 3 < ngroups)
            def _():
                wait_out(g0 + 1, buf_b, o_b)
                fire_gather(g0 + 3, buf_b, g_b)

        wait_out(ngroups - 1, buf_b, o_b)

    return gather_kernel(table, Xi)
